# Initial kernel scaffold; baseline (speedup 1.0000x reference)
import jax
import jax.numpy as jnp
from jax.experimental import pallas as pl


def kernel(raw, params):
    raise NotImplementedError("write your pallas kernel here")


# speedup vs baseline: 2.1998x; 2.1998x over previous
import jax
import jax.numpy as jnp
from jax.experimental import pallas as pl
from jax.experimental.pallas import tpu as pltpu

_N_KNOTS = 16
_N_SEG = _N_KNOTS + 1


def _spline_body(ys_ref, x_ref, o_ref):
    pid = pl.program_id(0)
    t = x_ref[0] * jnp.float32(_N_SEG)
    acc = None
    prev_d = jnp.float32(0.0)
    for i in range(_N_SEG):
        d = ys_ref[pid, i + 1] - ys_ref[pid, i]
        e = d - prev_d
        prev_d = d
        term = e * jnp.maximum(t - jnp.float32(i), jnp.float32(0.0))
        acc = term if acc is None else acc + term
    o_ref[0] = acc


def kernel(raw, params):
    B, C, H, W = raw.shape
    x = raw.reshape(B * C, H, W)
    ys_mid = params.reshape(B * C, _N_KNOTS)
    ys = jnp.concatenate(
        [
            jnp.zeros((B * C, 1), jnp.float32),
            ys_mid,
            jnp.ones((B * C, 1), jnp.float32),
        ],
        axis=1,
    )
    out = pl.pallas_call(
        _spline_body,
        grid=(B * C,),
        in_specs=[
            pl.BlockSpec(memory_space=pltpu.SMEM),
            pl.BlockSpec((1, H, W), lambda i: (i, 0, 0)),
        ],
        out_specs=pl.BlockSpec((1, H, W), lambda i: (i, 0, 0)),
        out_shape=jax.ShapeDtypeStruct((B * C, H, W), raw.dtype),
        compiler_params=pltpu.CompilerParams(
            dimension_semantics=("arbitrary",),
        ),
    )(ys, x)
    return out.reshape(B, C, H, W)
